# Initial kernel scaffold; baseline (speedup 1.0000x reference)
import jax
import jax.numpy as jnp
from jax.experimental import pallas as pl


def kernel(x):
    raise NotImplementedError("write your pallas kernel here")


# speedup vs baseline: 6.0877x; 6.0877x over previous
import functools
import math

import jax
import jax.numpy as jnp
import numpy as np
from jax import lax
from jax.experimental import pallas as pl
from jax.experimental.pallas import tpu as pltpu
from jax.experimental.pallas import tpu_sc as plsc

_SPARSITY = 0.05
_L = 16
_UNROLL = 8

_SIGN = np.int32(-(2**31))
_ONE = np.int32(1)
_ZERO = np.int32(0)


def _lane_sum(v):
    s = v[0]
    for lane in range(1, _L):
        s = s + v[lane]
    return s


def _lane_max(v):
    m = v[0]
    for lane in range(1, _L):
        m = jnp.maximum(m, v[lane])
    return m


def _bits2key(b):
    return jnp.where(b >= 0, b, jnp.bitwise_not(b) ^ _SIGN)


def _key2f_scalar(s):
    b = jnp.where(s >= 0, s, jnp.bitwise_not(s ^ _SIGN))
    return lax.bitcast_convert_type(b, jnp.float32)


def _f2key_scalar(f):
    b = lax.bitcast_convert_type(f, jnp.int32)
    return jnp.where(b >= 0, b, jnp.bitwise_not(b) ^ _SIGN)


@functools.partial(jax.jit, static_argnums=(1, 2, 3))
def _kwta_sc(xi, B, N, k):
    n_chunks = N // (_L * _UNROLL)

    mesh = plsc.VectorSubcoreMesh(core_axis_name="c", subcore_axis_name="s")

    @functools.partial(
        pl.kernel,
        mesh=mesh,
        out_type=jax.ShapeDtypeStruct((B, N), jnp.float32),
        scratch_types=[
            pltpu.VMEM((N,), jnp.int32),
            pltpu.VMEM((N,), jnp.float32),
        ],
    )
    def kwta(xi_hbm, out_hbm, kbuf, obuf):
        wid = lax.axis_index("s") * 2 + lax.axis_index("c")
        rows_per_w = B // 32

        def count_ge(c):
            def body(j, acc):
                base = j * (_L * _UNROLL)
                for u in range(_UNROLL):
                    v = kbuf[pl.ds(base + u * _L, _L)]
                    acc = acc + jnp.where(v >= c, _ONE, _ZERO)
                return acc
            acc = lax.fori_loop(0, n_chunks, body,
                                jnp.zeros((_L,), jnp.int32))
            return _lane_sum(acc)

        for r in range(rows_per_w):
            row = wid * rows_per_w + r
            pltpu.sync_copy(xi_hbm.at[row], kbuf)

            def conv_body(j, acc):
                base = j * (_L * _UNROLL)
                for u in range(_UNROLL):
                    b = kbuf[pl.ds(base + u * _L, _L)]
                    s = _bits2key(b)
                    kbuf[pl.ds(base + u * _L, _L)] = s
                    acc = acc + jnp.where(s >= 0, _ONE, _ZERO)
                return acc
            acc0 = lax.fori_loop(0, n_chunks, conv_body,
                                 jnp.zeros((_L,), jnp.int32))
            cnt0 = _lane_sum(acc0)
            p = jnp.where(cnt0 >= k, np.int32(0), _SIGN)

            def bit_body(i, p):
                c = p | (np.int32(1) << (np.int32(30) - i))
                cnt = count_ge(c)
                return jnp.where(cnt >= k, c, p)
            p = lax.fori_loop(0, 31, bit_body, p)

            def low_body(j, carry):
                acc, mx = carry
                base = j * (_L * _UNROLL)
                for u in range(_UNROLL):
                    v = kbuf[pl.ds(base + u * _L, _L)]
                    ge = v >= p
                    acc = acc + jnp.where(ge, _ONE, _ZERO)
                    mx = jnp.maximum(mx, jnp.where(ge, _SIGN, v))
                return acc, mx
            accg, mxv = lax.fori_loop(
                0, n_chunks, low_body,
                (jnp.zeros((_L,), jnp.int32),
                 jnp.full((_L,), _SIGN, jnp.int32)))
            cnt_ge = _lane_sum(accg)
            max_low = _lane_max(mxv)
            s2 = jnp.where(cnt_ge >= k + 1, p, max_low)

            thr = (_key2f_scalar(p) + _key2f_scalar(s2)) * np.float32(0.5)
            tkey = jnp.where(thr == np.float32(0.0), np.int32(0),
                             _f2key_scalar(thr))

            def mask_body(j, _):
                base = j * (_L * _UNROLL)
                for u in range(_UNROLL):
                    v = kbuf[pl.ds(base + u * _L, _L)]
                    obuf[pl.ds(base + u * _L, _L)] = jnp.where(
                        v > tkey, np.float32(1.0), np.float32(0.0))
                return 0
            lax.fori_loop(0, n_chunks, mask_body, 0)

            pltpu.sync_copy(obuf, out_hbm.at[row])

    return kwta(xi)


def kernel(x):
    B, N = x.shape
    k = math.ceil(_SPARSITY * N)
    if k == N:
        k -= 1
    xi = lax.bitcast_convert_type(x, jnp.int32)
    return _kwta_sc(xi, B, N, k)
